# Initial kernel scaffold; baseline (speedup 1.0000x reference)
import jax
import jax.numpy as jnp
from jax.experimental import pallas as pl


def kernel(feats, results):
    raise NotImplementedError("write your pallas kernel here")


# speedup vs baseline: 3.8957x; 3.8957x over previous
import functools

import jax
import jax.numpy as jnp
from jax import lax
from jax.experimental import pallas as pl
from jax.experimental.pallas import tpu as pltpu
from jax.experimental.pallas import tpu_sc as plsc

N_IN = 100000
N_OUT = 50000
K = 8
C = 128
LANES = 16

CH = 80
NCH = N_OUT // CH
NC = 2
NS = 16
NW = NC * NS


def kernel(feats, results):
    mesh = plsc.VectorSubcoreMesh(core_axis_name="c", subcore_axis_name="s")

    @functools.partial(
        pl.kernel,
        mesh=mesh,
        out_type=jax.ShapeDtypeStruct((N_OUT, C), jnp.float32),
        scratch_types=[
            pltpu.VMEM((K, CH), jnp.int32),
            pltpu.VMEM((K, CH, C), jnp.float32),
            pltpu.VMEM((CH, C), jnp.float32),
            pltpu.SemaphoreType.DMA,
            pltpu.SemaphoreType.DMA,
        ],
    )
    def run(feats_hbm, res_hbm, out_hbm, idx_v, rows_v, out_v, gsem, osem):
        wid = lax.axis_index("s") * NC + lax.axis_index("c")
        n_chunks = (NCH - 1 - wid) // NW + 1

        def chunk_body(i, carry):
            base = (wid + i * NW) * CH
            for k in range(K):
                pltpu.sync_copy(
                    res_hbm.at[pl.ds(k * N_OUT + base, CH)], idx_v.at[k]
                )
            copies = [
                pltpu.async_copy(feats_hbm.at[idx_v.at[k]], rows_v.at[k], gsem)
                for k in range(K)
            ]
            for cp in copies:
                cp.wait()

            def row_body(r, rcarry):
                for g in range(C // LANES):
                    sl = pl.ds(g * LANES, LANES)
                    acc = rows_v[0, r, sl]
                    for k in range(1, K):
                        acc = acc + rows_v[k, r, sl]
                    out_v[r, sl] = acc * 0.125
                return rcarry

            lax.fori_loop(0, CH, row_body, 0)
            pltpu.async_copy(out_v, out_hbm.at[pl.ds(base, CH)], osem).wait()
            return carry

        lax.fori_loop(0, n_chunks, chunk_body, 0)

    return run(feats, results.reshape(-1))


# speedup vs baseline: 8.0652x; 2.0703x over previous
import functools

import jax
import jax.numpy as jnp
from jax import lax
from jax.experimental import pallas as pl
from jax.experimental.pallas import tpu as pltpu
from jax.experimental.pallas import tpu_sc as plsc

N_IN = 100000
N_OUT = 50000
K = 8
C = 128
LANES = 16
GRP = C // LANES

CH = 80
NCH = N_OUT // CH
NC = 2
NS = 16
NW = NC * NS
MAXJ = (NCH + NW - 1) // NW


def kernel(feats, results):
    mesh = plsc.VectorSubcoreMesh(core_axis_name="c", subcore_axis_name="s")

    @functools.partial(
        pl.kernel,
        mesh=mesh,
        out_type=jax.ShapeDtypeStruct((N_OUT, C), jnp.float32),
        scratch_types=[
            pltpu.VMEM((2, K, CH), jnp.int32),
            pltpu.VMEM((2, CH, C), jnp.float32),
            pltpu.VMEM((2, CH, C), jnp.float32),
            pltpu.SemaphoreType.DMA,
            pltpu.SemaphoreType.DMA,
            pltpu.SemaphoreType.DMA,
            pltpu.SemaphoreType.DMA,
            pltpu.SemaphoreType.DMA,
            pltpu.SemaphoreType.DMA,
        ],
    )
    def run(feats_hbm, res_hbm, out_hbm, idx, acc, outb,
            isem0, isem1, gsem0, gsem1, osem0, osem1):
        wid = lax.axis_index("s") * NC + lax.axis_index("c")
        lo = wid * NCH // NW
        hi = (wid + 1) * NCH // NW

        cjs = [lax.min(lo + j, hi - 1) for j in range(MAXJ)]

        isem = (isem0, isem1)
        gsem = (gsem0, gsem1)
        osem = (osem0, osem1)

        zeros = jnp.zeros((LANES,), jnp.float32)

        def zero_body(r, carry):
            for g in range(GRP):
                sl = pl.ds(g * LANES, LANES)
                acc[0, r, sl] = zeros
                acc[1, r, sl] = zeros
            return carry

        lax.fori_loop(0, CH, zero_body, 0)

        def fire_idx(cj, b):
            return [
                pltpu.async_copy(
                    res_hbm.at[pl.ds(k * N_OUT + cj * CH, CH)],
                    idx.at[b, k],
                    isem[b],
                )
                for k in range(K)
            ]

        def fire_gather(b):
            return [
                pltpu.async_copy(
                    feats_hbm.at[idx.at[b, k]],
                    acc.at[b],
                    gsem[b],
                    add=True,
                )
                for k in range(K)
            ]

        def scale_out(cj, b):
            def sbody(r, carry):
                for g in range(GRP):
                    sl = pl.ds(g * LANES, LANES)
                    outb[b, r, sl] = acc[b, r, sl] * 0.125
                    acc[b, r, sl] = zeros
                return carry

            lax.fori_loop(0, CH, sbody, 0)
            return pltpu.async_copy(
                outb.at[b],
                out_hbm.at[pl.ds(cj * CH, CH)],
                osem[b],
            )

        pend_i = [None, None]
        pend_g = [None, None]
        pend_o = [None, None]

        pend_i[0] = fire_idx(cjs[0], 0)
        pend_i[1] = fire_idx(cjs[1], 1)
        for cp in pend_i[0]:
            cp.wait()
        pend_g[0] = fire_gather(0)

        for j in range(MAXJ):
            p = j & 1
            q = 1 - p

            if j + 1 < MAXJ:
                for cp in pend_i[q]:
                    cp.wait()
                pend_g[q] = fire_gather(q)

            for cp in pend_g[p]:
                cp.wait()
            if j + 2 < MAXJ:
                pend_i[p] = fire_idx(cjs[j + 2], p)
            if j >= 2:
                pend_o[p].wait()
            pend_o[p] = scale_out(cjs[j], p)

        pend_o[0].wait()
        pend_o[1].wait()

    return run(feats, results.reshape(-1))
